# baseline (speedup 1.0000x reference)
import jax
import jax.numpy as jnp
from jax import lax
from jax.experimental import pallas as pl
from jax.experimental.pallas import tpu as pltpu
from jax.experimental.pallas import tpu_sc as plsc

N = 10000
E = 320000
D = 128
G = 64

NC = 2
NS = 16
NW = NC * NS

EB = 128
JP = 40
NP = 2
JW = JP * NP
E_PAD = NW * JW * EB
NZ = 10112
RPS = NZ // NS


def _sc_aggr_body(h_hbm, src_hbm, dst_hbm, zero_hbm, out_hbm,
                  aggr, sidx, didx, rows0, rows1, sem0, sem1, semz):
    c = lax.axis_index("c")
    s = lax.axis_index("s")
    wid = c * NS + s
    row0 = pl.multiple_of(s * RPS, 8)
    zcp = pltpu.async_copy(zero_hbm.at[pl.ds(row0, RPS)],
                           aggr.at[pl.ds(row0, RPS)], semz)
    pltpu.sync_copy(src_hbm.at[wid, 0], sidx)
    pltpu.sync_copy(dst_hbm.at[wid, 0], didx)
    zcp.wait()
    plsc.subcore_barrier()

    for p in range(NP):
        @pl.loop(0, JP)
        def _(j):
            pltpu.sync_copy(h_hbm.at[sidx.at[j]], rows0)
            pltpu.sync_copy(rows0, aggr.at[didx.at[j]], add=True)

        if p + 1 < NP:
            pltpu.sync_copy(src_hbm.at[wid, p + 1], sidx)
            pltpu.sync_copy(dst_hbm.at[wid, p + 1], didx)

    plsc.subcore_barrier()
    pltpu.sync_copy(aggr.at[pl.ds(row0, RPS)],
                    out_hbm.at[c, pl.ds(row0, RPS)])


@jax.jit
def _sc_aggr(h, src_r, dst_r, zeros):
    mesh = plsc.VectorSubcoreMesh(core_axis_name="c", subcore_axis_name="s")
    return pl.kernel(
        _sc_aggr_body,
        out_type=jax.ShapeDtypeStruct((NC, NZ, D), jnp.float32),
        mesh=mesh,
        scratch_types=[
            pltpu.VMEM_SHARED((NZ, D), jnp.float32),
            pltpu.VMEM((JP + 1, EB), jnp.int32),
            pltpu.VMEM((JP + 1, EB), jnp.int32),
            pltpu.VMEM((EB, D), jnp.float32),
            pltpu.VMEM((EB, D), jnp.float32),
            pltpu.SemaphoreType.DMA,
            pltpu.SemaphoreType.DMA,
            pltpu.SemaphoreType.DMA,
        ],
    )(h, src_r, dst_r, zeros)


def _mlp_body(x_ref, a0_ref, a1_ref, w1_ref, b1_ref, w2_ref, b2_ref, o_ref):
    z = x_ref[...] + a0_ref[...] + a1_ref[...]
    z1 = jnp.maximum(
        jnp.dot(z, w1_ref[...], preferred_element_type=jnp.float32)
        + b1_ref[...], 0.0)
    o_ref[...] = (jnp.dot(z1, w2_ref[...], preferred_element_type=jnp.float32)
                  + b2_ref[...])


def _mlp_pool_body(x_ref, a0_ref, a1_ref, w1_ref, b1_ref, w2_ref, b2_ref,
                   batch_ref, o_ref):
    i = pl.program_id(0)
    z = x_ref[...] + a0_ref[...] + a1_ref[...]
    z1 = jnp.maximum(
        jnp.dot(z, w1_ref[...], preferred_element_type=jnp.float32)
        + b1_ref[...], 0.0)
    h3 = (jnp.dot(z1, w2_ref[...], preferred_element_type=jnp.float32)
          + b2_ref[...])
    ids = batch_ref[0, 0]
    mask = (jax.lax.broadcasted_iota(jnp.int32, (G, ids.shape[0]), 0)
            == ids[None, :]).astype(jnp.float32)
    pooled = jnp.dot(mask, h3, preferred_element_type=jnp.float32)

    @pl.when(i == 0)
    def _():
        o_ref[...] = jnp.zeros_like(o_ref)

    o_ref[...] += pooled


_RB = 2000
_NB = N // _RB

_TC_SPECS = [
    pl.BlockSpec((_RB, D), lambda i: (i, 0)),
    pl.BlockSpec((_RB, D), lambda i: (i, 0)),
    pl.BlockSpec((_RB, D), lambda i: (i, 0)),
    pl.BlockSpec((D, D), lambda i: (0, 0)),
    pl.BlockSpec((1, D), lambda i: (0, 0)),
    pl.BlockSpec((D, D), lambda i: (0, 0)),
    pl.BlockSpec((1, D), lambda i: (0, 0)),
]


@jax.jit
def _tc_mlp(h, a0, a1, w1, b1, w2, b2):
    return pl.pallas_call(
        _mlp_body,
        grid=(_NB,),
        in_specs=_TC_SPECS,
        out_specs=pl.BlockSpec((_RB, D), lambda i: (i, 0)),
        out_shape=jax.ShapeDtypeStruct((N, D), jnp.float32),
    )(h, a0, a1, w1, b1.reshape(1, D), w2, b2.reshape(1, D))


@jax.jit
def _tc_mlp_pool(h, a0, a1, w1, b1, w2, b2, batch_r):
    return pl.pallas_call(
        _mlp_pool_body,
        grid=(_NB,),
        in_specs=_TC_SPECS + [pl.BlockSpec((1, 1, _RB), lambda i: (i, 0, 0))],
        out_specs=pl.BlockSpec((G, D), lambda i: (0, 0)),
        out_shape=jax.ShapeDtypeStruct((G, D), jnp.float32),
    )(h, a0, a1, w1, b1.reshape(1, D), w2, b2.reshape(1, D), batch_r)


def _stage_idx(v, fill):
    pad = E_PAD - E
    r = jnp.concatenate([v, jnp.full((pad,), fill, jnp.int32)])
    r = r.reshape(NW, JW, EB)
    r = jnp.concatenate([r, jnp.full((NW, 1, EB), fill, jnp.int32)], 1)
    passes = [r[:, p * JP:p * JP + JP + 1] for p in range(NP)]
    return jnp.stack(passes, axis=1)


def kernel(x, edge_index, batch, W1_0, b1_0, W2_0, b2_0, W1_1, b1_1, W2_1,
           b2_1, W1_2, b1_2, W2_2, b2_2):
    src_r = _stage_idx(edge_index[0], 0)
    dst_r = _stage_idx(edge_index[1], N)
    zeros = jnp.zeros((NZ, D), jnp.float32)
    batch_r = batch.reshape(_NB, 1, _RB)

    params = [(W1_0, b1_0, W2_0, b2_0), (W1_1, b1_1, W2_1, b2_1),
              (W1_2, b1_2, W2_2, b2_2)]
    h = x
    out = None
    for l, (w1, b1, w2, b2) in enumerate(params):
        parts = _sc_aggr(h, src_r, dst_r, zeros)
        a0 = parts[0, :N]
        a1 = parts[1, :N]
        if l < 2:
            h = _tc_mlp(h, a0, a1, w1, b1, w2, b2)
        else:
            out = _tc_mlp_pool(h, a0, a1, w1, b1, w2, b2, batch_r)
    return out


# speedup vs baseline: 2.6573x; 2.6573x over previous
import jax
import jax.numpy as jnp
from jax import lax
from jax.experimental import pallas as pl
from jax.experimental.pallas import tpu as pltpu
from jax.experimental.pallas import tpu_sc as plsc

N = 10000
E = 320000
D = 128
G = 64

NC = 2
NS = 16
NW = NC * NS

EB = 128
JW = 80
E_PAD = NW * JW * EB
NZ = 10112
NTRASH = NZ - N
RPS = NZ // NS


def _sc_aggr_body(h_hbm, src_hbm, dst_hbm, zero_hbm, out_hbm,
                  aggr, sidx, didx, rows0, rows1, sem0, sem1, semz):
    c = lax.axis_index("c")
    s = lax.axis_index("s")
    wid = c * NS + s
    row0 = pl.multiple_of(s * RPS, 8)
    zcp = pltpu.async_copy(zero_hbm.at[pl.ds(row0, RPS)],
                           aggr.at[pl.ds(row0, RPS)], semz)
    pltpu.sync_copy(src_hbm.at[wid], sidx)
    pltpu.sync_copy(dst_hbm.at[wid], didx)
    zcp.wait()
    plsc.subcore_barrier()

    @pl.loop(0, JW)
    def _(j):
        pltpu.sync_copy(h_hbm.at[sidx.at[j]], rows0)
        pltpu.sync_copy(rows0, aggr.at[didx.at[j]], add=True)

    plsc.subcore_barrier()
    pltpu.sync_copy(aggr.at[pl.ds(row0, RPS)],
                    out_hbm.at[c, pl.ds(row0, RPS)])


@jax.jit
def _sc_aggr(h, src_r, dst_r, zeros):
    mesh = plsc.VectorSubcoreMesh(core_axis_name="c", subcore_axis_name="s")
    return pl.kernel(
        _sc_aggr_body,
        out_type=jax.ShapeDtypeStruct((NC, NZ, D), jnp.float32),
        mesh=mesh,
        scratch_types=[
            pltpu.VMEM_SHARED((NZ, D), jnp.float32),
            pltpu.VMEM((JW, EB), jnp.int32),
            pltpu.VMEM((JW, EB), jnp.int32),
            pltpu.VMEM((EB, D), jnp.float32),
            pltpu.VMEM((EB, D), jnp.float32),
            pltpu.SemaphoreType.DMA,
            pltpu.SemaphoreType.DMA,
            pltpu.SemaphoreType.DMA,
        ],
    )(h, src_r, dst_r, zeros)


def _mlp_body(x_ref, a0_ref, a1_ref, w1_ref, b1_ref, w2_ref, b2_ref, o_ref):
    z = x_ref[...] + a0_ref[...] + a1_ref[...]
    z1 = jnp.maximum(
        jnp.dot(z, w1_ref[...], preferred_element_type=jnp.float32)
        + b1_ref[...], 0.0)
    o_ref[...] = (jnp.dot(z1, w2_ref[...], preferred_element_type=jnp.float32)
                  + b2_ref[...])


def _mlp_pool_body(x_ref, a0_ref, a1_ref, w1_ref, b1_ref, w2_ref, b2_ref,
                   batch_ref, o_ref):
    i = pl.program_id(0)
    z = x_ref[...] + a0_ref[...] + a1_ref[...]
    z1 = jnp.maximum(
        jnp.dot(z, w1_ref[...], preferred_element_type=jnp.float32)
        + b1_ref[...], 0.0)
    h3 = (jnp.dot(z1, w2_ref[...], preferred_element_type=jnp.float32)
          + b2_ref[...])
    ids = batch_ref[0, 0]
    mask = (jax.lax.broadcasted_iota(jnp.int32, (G, ids.shape[0]), 0)
            == ids[None, :]).astype(jnp.float32)
    pooled = jnp.dot(mask, h3, preferred_element_type=jnp.float32)

    @pl.when(i == 0)
    def _():
        o_ref[...] = jnp.zeros_like(o_ref)

    o_ref[...] += pooled


_RB = 2000
_NB = N // _RB

_TC_SPECS = [
    pl.BlockSpec((_RB, D), lambda i: (i, 0)),
    pl.BlockSpec((_RB, D), lambda i: (i, 0)),
    pl.BlockSpec((_RB, D), lambda i: (i, 0)),
    pl.BlockSpec((D, D), lambda i: (0, 0)),
    pl.BlockSpec((1, D), lambda i: (0, 0)),
    pl.BlockSpec((D, D), lambda i: (0, 0)),
    pl.BlockSpec((1, D), lambda i: (0, 0)),
]


@jax.jit
def _tc_mlp(h, a0, a1, w1, b1, w2, b2):
    return pl.pallas_call(
        _mlp_body,
        grid=(_NB,),
        in_specs=_TC_SPECS,
        out_specs=pl.BlockSpec((_RB, D), lambda i: (i, 0)),
        out_shape=jax.ShapeDtypeStruct((N, D), jnp.float32),
    )(h, a0, a1, w1, b1.reshape(1, D), w2, b2.reshape(1, D))


@jax.jit
def _tc_mlp_pool(h, a0, a1, w1, b1, w2, b2, batch_r):
    return pl.pallas_call(
        _mlp_pool_body,
        grid=(_NB,),
        in_specs=_TC_SPECS + [pl.BlockSpec((1, 1, _RB), lambda i: (i, 0, 0))],
        out_specs=pl.BlockSpec((G, D), lambda i: (0, 0)),
        out_shape=jax.ShapeDtypeStruct((G, D), jnp.float32),
    )(h, a0, a1, w1, b1.reshape(1, D), w2, b2.reshape(1, D), batch_r)


def _stage_idx(v, fills):
    return jnp.concatenate([v, fills]).reshape(NW, JW, EB)


def kernel(x, edge_index, batch, W1_0, b1_0, W2_0, b2_0, W1_1, b1_1, W2_1,
           b2_1, W1_2, b1_2, W2_2, b2_2):
    pad = jnp.arange(E_PAD - E, dtype=jnp.int32)
    src_r = _stage_idx(edge_index[0], pad % N)
    dst_r = _stage_idx(edge_index[1], N + pad % NTRASH)
    zeros = jnp.zeros((NZ, D), jnp.float32)
    batch_r = batch.reshape(_NB, 1, _RB)

    params = [(W1_0, b1_0, W2_0, b2_0), (W1_1, b1_1, W2_1, b2_1),
              (W1_2, b1_2, W2_2, b2_2)]
    h = x
    out = None
    for l, (w1, b1, w2, b2) in enumerate(params):
        parts = _sc_aggr(h, src_r, dst_r, zeros)
        a0 = parts[0, :N]
        a1 = parts[1, :N]
        if l < 2:
            h = _tc_mlp(h, a0, a1, w1, b1, w2, b2)
        else:
            out = _tc_mlp_pool(h, a0, a1, w1, b1, w2, b2, batch_r)
    return out


# speedup vs baseline: 3.3996x; 1.2794x over previous
import jax
import jax.numpy as jnp
from jax import lax
from jax.experimental import pallas as pl
from jax.experimental.pallas import tpu as pltpu
from jax.experimental.pallas import tpu_sc as plsc

N = 10000
E = 320000
D = 128
G = 64

NC = 2
NS = 16
NW = NC * NS

EB = 128
JP = 40
NP = 2
JW = JP * NP
E_PAD = NW * JW * EB
NZ = 10112
NTRASH = NZ - N
RPS = NZ // NS


def _sc_aggr_body(h_hbm, src_hbm, dst_hbm, zero_hbm, out_hbm,
                  aggr, sidx, didx, rows0, rows1, sem0, sem1, semz):
    c = lax.axis_index("c")
    s = lax.axis_index("s")
    wid = c * NS + s
    row0 = pl.multiple_of(s * RPS, 8)
    zcp = pltpu.async_copy(zero_hbm.at[pl.ds(row0, RPS)],
                           aggr.at[pl.ds(row0, RPS)], semz)
    pltpu.sync_copy(src_hbm.at[wid, 0], sidx)
    pltpu.sync_copy(dst_hbm.at[wid, 0], didx)
    zcp.wait()
    plsc.subcore_barrier()

    for p in range(NP):
        if p == 0:
            pltpu.sync_copy(h_hbm.at[sidx.at[0]], rows0)

        @pl.loop(0, JP, step=2)
        def _(j):
            s0 = pltpu.async_copy(rows0, aggr.at[didx.at[j]], sem0,
                                  add=True)
            pltpu.sync_copy(h_hbm.at[sidx.at[j + 1]], rows1)
            s0.wait()
            s1 = pltpu.async_copy(rows1, aggr.at[didx.at[j + 1]], sem1,
                                  add=True)
            pltpu.sync_copy(h_hbm.at[sidx.at[j + 2]], rows0)
            s1.wait()

        if p + 1 < NP:
            pltpu.sync_copy(src_hbm.at[wid, p + 1], sidx)
            pltpu.sync_copy(dst_hbm.at[wid, p + 1], didx)

    plsc.subcore_barrier()
    pltpu.sync_copy(aggr.at[pl.ds(row0, RPS)],
                    out_hbm.at[c, pl.ds(row0, RPS)])


@jax.jit
def _sc_aggr(h, src_r, dst_r, zeros):
    mesh = plsc.VectorSubcoreMesh(core_axis_name="c", subcore_axis_name="s")
    return pl.kernel(
        _sc_aggr_body,
        out_type=jax.ShapeDtypeStruct((NC, NZ, D), jnp.float32),
        mesh=mesh,
        scratch_types=[
            pltpu.VMEM_SHARED((NZ, D), jnp.float32),
            pltpu.VMEM((JP + 1, EB), jnp.int32),
            pltpu.VMEM((JP + 1, EB), jnp.int32),
            pltpu.VMEM((EB, D), jnp.float32),
            pltpu.VMEM((EB, D), jnp.float32),
            pltpu.SemaphoreType.DMA,
            pltpu.SemaphoreType.DMA,
            pltpu.SemaphoreType.DMA,
        ],
    )(h, src_r, dst_r, zeros)


def _mlp_body(x_ref, a0_ref, a1_ref, w1_ref, b1_ref, w2_ref, b2_ref, o_ref):
    z = x_ref[...] + a0_ref[...] + a1_ref[...]
    z1 = jnp.maximum(
        jnp.dot(z, w1_ref[...], preferred_element_type=jnp.float32)
        + b1_ref[...], 0.0)
    o_ref[...] = (jnp.dot(z1, w2_ref[...], preferred_element_type=jnp.float32)
                  + b2_ref[...])


def _mlp_pool_body(x_ref, a0_ref, a1_ref, w1_ref, b1_ref, w2_ref, b2_ref,
                   batch_ref, o_ref):
    i = pl.program_id(0)
    z = x_ref[...] + a0_ref[...] + a1_ref[...]
    z1 = jnp.maximum(
        jnp.dot(z, w1_ref[...], preferred_element_type=jnp.float32)
        + b1_ref[...], 0.0)
    h3 = (jnp.dot(z1, w2_ref[...], preferred_element_type=jnp.float32)
          + b2_ref[...])
    ids = batch_ref[0, 0]
    mask = (jax.lax.broadcasted_iota(jnp.int32, (G, ids.shape[0]), 0)
            == ids[None, :]).astype(jnp.float32)
    pooled = jnp.dot(mask, h3, preferred_element_type=jnp.float32)

    @pl.when(i == 0)
    def _():
        o_ref[...] = jnp.zeros_like(o_ref)

    o_ref[...] += pooled


_RB = 2000
_NB = N // _RB

_TC_SPECS = [
    pl.BlockSpec((_RB, D), lambda i: (i, 0)),
    pl.BlockSpec((_RB, D), lambda i: (i, 0)),
    pl.BlockSpec((_RB, D), lambda i: (i, 0)),
    pl.BlockSpec((D, D), lambda i: (0, 0)),
    pl.BlockSpec((1, D), lambda i: (0, 0)),
    pl.BlockSpec((D, D), lambda i: (0, 0)),
    pl.BlockSpec((1, D), lambda i: (0, 0)),
]


@jax.jit
def _tc_mlp(h, a0, a1, w1, b1, w2, b2):
    return pl.pallas_call(
        _mlp_body,
        grid=(_NB,),
        in_specs=_TC_SPECS,
        out_specs=pl.BlockSpec((_RB, D), lambda i: (i, 0)),
        out_shape=jax.ShapeDtypeStruct((N, D), jnp.float32),
    )(h, a0, a1, w1, b1.reshape(1, D), w2, b2.reshape(1, D))


@jax.jit
def _tc_mlp_pool(h, a0, a1, w1, b1, w2, b2, batch_r):
    return pl.pallas_call(
        _mlp_pool_body,
        grid=(_NB,),
        in_specs=_TC_SPECS + [pl.BlockSpec((1, 1, _RB), lambda i: (i, 0, 0))],
        out_specs=pl.BlockSpec((G, D), lambda i: (0, 0)),
        out_shape=jax.ShapeDtypeStruct((G, D), jnp.float32),
    )(h, a0, a1, w1, b1.reshape(1, D), w2, b2.reshape(1, D), batch_r)


def _stage_idx(v, fills, tail):
    r = jnp.concatenate([v, fills]).reshape(NW, JW, EB)
    r = jnp.concatenate([r, tail.reshape(NW, 1, EB)], 1)
    passes = [r[:, p * JP:p * JP + JP + 1] for p in range(NP)]
    return jnp.stack(passes, axis=1)


def kernel(x, edge_index, batch, W1_0, b1_0, W2_0, b2_0, W1_1, b1_1, W2_1,
           b2_1, W1_2, b1_2, W2_2, b2_2):
    pad = jnp.arange(E_PAD - E, dtype=jnp.int32)
    tail = jnp.arange(NW * EB, dtype=jnp.int32)
    src_r = _stage_idx(edge_index[0], pad % N, tail % N)
    dst_r = _stage_idx(edge_index[1], N + pad % NTRASH, N + tail % NTRASH)
    zeros = jnp.zeros((NZ, D), jnp.float32)
    batch_r = batch.reshape(_NB, 1, _RB)

    params = [(W1_0, b1_0, W2_0, b2_0), (W1_1, b1_1, W2_1, b2_1),
              (W1_2, b1_2, W2_2, b2_2)]
    h = x
    out = None
    for l, (w1, b1, w2, b2) in enumerate(params):
        parts = _sc_aggr(h, src_r, dst_r, zeros)
        a0 = parts[0, :N]
        a1 = parts[1, :N]
        if l < 2:
            h = _tc_mlp(h, a0, a1, w1, b1, w2, b2)
        else:
            out = _tc_mlp_pool(h, a0, a1, w1, b1, w2, b2, batch_r)
    return out


# speedup vs baseline: 3.4007x; 1.0003x over previous
import jax
import jax.numpy as jnp
from jax import lax
from jax.experimental import pallas as pl
from jax.experimental.pallas import tpu as pltpu
from jax.experimental.pallas import tpu_sc as plsc

N = 10000
E = 320000
D = 128
G = 64

NC = 2
NS = 16
NW = NC * NS

EB = 128
JP = 40
NP = 2
JW = JP * NP
E_PAD = NW * JW * EB
NZ = 10112
NTRASH = NZ - N
RPS = NZ // NS


def _sc_aggr_body(h_hbm, src_hbm, dst_hbm, zero_hbm, out_hbm,
                  aggr, sidx, didx, rows0, rows1, sem0, sem1, semz):
    c = lax.axis_index("c")
    s = lax.axis_index("s")
    wid = c * NS + s
    row0 = pl.multiple_of(s * RPS, 8)
    zcp = pltpu.async_copy(zero_hbm.at[pl.ds(row0, RPS)],
                           aggr.at[pl.ds(row0, RPS)], semz)
    pltpu.sync_copy(src_hbm.at[wid, 0], sidx)
    pltpu.sync_copy(dst_hbm.at[wid, 0], didx)
    zcp.wait()
    plsc.subcore_barrier()

    for p in range(NP):
        if p == 0:
            pltpu.sync_copy(h_hbm.at[sidx.at[0]], rows0)

        @pl.loop(0, JP, step=2)
        def _(j):
            s0 = pltpu.async_copy(rows0, aggr.at[didx.at[j]], sem0,
                                  add=True)
            pltpu.sync_copy(h_hbm.at[sidx.at[j + 1]], rows1)
            s1 = pltpu.async_copy(rows1, aggr.at[didx.at[j + 1]], sem1,
                                  add=True)
            s0.wait()
            pltpu.sync_copy(h_hbm.at[sidx.at[j + 2]], rows0)
            s1.wait()

        if p + 1 < NP:
            pltpu.sync_copy(src_hbm.at[wid, p + 1], sidx)
            pltpu.sync_copy(dst_hbm.at[wid, p + 1], didx)

    plsc.subcore_barrier()
    pltpu.sync_copy(aggr.at[pl.ds(row0, RPS)],
                    out_hbm.at[c, pl.ds(row0, RPS)])


@jax.jit
def _sc_aggr(h, src_r, dst_r, zeros):
    mesh = plsc.VectorSubcoreMesh(core_axis_name="c", subcore_axis_name="s")
    return pl.kernel(
        _sc_aggr_body,
        out_type=jax.ShapeDtypeStruct((NC, NZ, D), jnp.float32),
        mesh=mesh,
        scratch_types=[
            pltpu.VMEM_SHARED((NZ, D), jnp.float32),
            pltpu.VMEM((JP + 1, EB), jnp.int32),
            pltpu.VMEM((JP + 1, EB), jnp.int32),
            pltpu.VMEM((EB, D), jnp.float32),
            pltpu.VMEM((EB, D), jnp.float32),
            pltpu.SemaphoreType.DMA,
            pltpu.SemaphoreType.DMA,
            pltpu.SemaphoreType.DMA,
        ],
    )(h, src_r, dst_r, zeros)


def _mlp_body(x_ref, a0_ref, a1_ref, w1_ref, b1_ref, w2_ref, b2_ref, o_ref):
    z = x_ref[...] + a0_ref[...] + a1_ref[...]
    z1 = jnp.maximum(
        jnp.dot(z, w1_ref[...], preferred_element_type=jnp.float32)
        + b1_ref[...], 0.0)
    o_ref[...] = (jnp.dot(z1, w2_ref[...], preferred_element_type=jnp.float32)
                  + b2_ref[...])


def _mlp_pool_body(x_ref, a0_ref, a1_ref, w1_ref, b1_ref, w2_ref, b2_ref,
                   batch_ref, o_ref):
    i = pl.program_id(0)
    z = x_ref[...] + a0_ref[...] + a1_ref[...]
    z1 = jnp.maximum(
        jnp.dot(z, w1_ref[...], preferred_element_type=jnp.float32)
        + b1_ref[...], 0.0)
    h3 = (jnp.dot(z1, w2_ref[...], preferred_element_type=jnp.float32)
          + b2_ref[...])
    ids = batch_ref[0, 0]
    mask = (jax.lax.broadcasted_iota(jnp.int32, (G, ids.shape[0]), 0)
            == ids[None, :]).astype(jnp.float32)
    pooled = jnp.dot(mask, h3, preferred_element_type=jnp.float32)

    @pl.when(i == 0)
    def _():
        o_ref[...] = jnp.zeros_like(o_ref)

    o_ref[...] += pooled


_RB = 2000
_NB = N // _RB

_TC_SPECS = [
    pl.BlockSpec((_RB, D), lambda i: (i, 0)),
    pl.BlockSpec((_RB, D), lambda i: (i, 0)),
    pl.BlockSpec((_RB, D), lambda i: (i, 0)),
    pl.BlockSpec((D, D), lambda i: (0, 0)),
    pl.BlockSpec((1, D), lambda i: (0, 0)),
    pl.BlockSpec((D, D), lambda i: (0, 0)),
    pl.BlockSpec((1, D), lambda i: (0, 0)),
]


@jax.jit
def _tc_mlp(h, a0, a1, w1, b1, w2, b2):
    return pl.pallas_call(
        _mlp_body,
        grid=(_NB,),
        in_specs=_TC_SPECS,
        out_specs=pl.BlockSpec((_RB, D), lambda i: (i, 0)),
        out_shape=jax.ShapeDtypeStruct((N, D), jnp.float32),
    )(h, a0, a1, w1, b1.reshape(1, D), w2, b2.reshape(1, D))


@jax.jit
def _tc_mlp_pool(h, a0, a1, w1, b1, w2, b2, batch_r):
    return pl.pallas_call(
        _mlp_pool_body,
        grid=(_NB,),
        in_specs=_TC_SPECS + [pl.BlockSpec((1, 1, _RB), lambda i: (i, 0, 0))],
        out_specs=pl.BlockSpec((G, D), lambda i: (0, 0)),
        out_shape=jax.ShapeDtypeStruct((G, D), jnp.float32),
    )(h, a0, a1, w1, b1.reshape(1, D), w2, b2.reshape(1, D), batch_r)


def _stage_idx(v, fills, tail):
    r = jnp.concatenate([v, fills]).reshape(NW, JW, EB)
    r = jnp.concatenate([r, tail.reshape(NW, 1, EB)], 1)
    passes = [r[:, p * JP:p * JP + JP + 1] for p in range(NP)]
    return jnp.stack(passes, axis=1)


def kernel(x, edge_index, batch, W1_0, b1_0, W2_0, b2_0, W1_1, b1_1, W2_1,
           b2_1, W1_2, b1_2, W2_2, b2_2):
    pad = jnp.arange(E_PAD - E, dtype=jnp.int32)
    tail = jnp.arange(NW * EB, dtype=jnp.int32)
    src_r = _stage_idx(edge_index[0], pad % N, tail % N)
    dst_r = _stage_idx(edge_index[1], N + pad % NTRASH, N + tail % NTRASH)
    zeros = jnp.zeros((NZ, D), jnp.float32)
    batch_r = batch.reshape(_NB, 1, _RB)

    params = [(W1_0, b1_0, W2_0, b2_0), (W1_1, b1_1, W2_1, b2_1),
              (W1_2, b1_2, W2_2, b2_2)]
    h = x
    out = None
    for l, (w1, b1, w2, b2) in enumerate(params):
        parts = _sc_aggr(h, src_r, dst_r, zeros)
        a0 = parts[0, :N]
        a1 = parts[1, :N]
        if l < 2:
            h = _tc_mlp(h, a0, a1, w1, b1, w2, b2)
        else:
            out = _tc_mlp_pool(h, a0, a1, w1, b1, w2, b2, batch_r)
    return out


# speedup vs baseline: 3.5688x; 1.0494x over previous
import jax
import jax.numpy as jnp
from jax import lax
from jax.experimental import pallas as pl
from jax.experimental.pallas import tpu as pltpu
from jax.experimental.pallas import tpu_sc as plsc

N = 10000
E = 320000
D = 128
G = 64

NC = 2
NS = 16
NW = NC * NS

EB = 128
JP = 40
NP = 2
JW = JP * NP
E_PAD = NW * JW * EB
NZ = 10112
NTRASH = NZ - N
RPS = NZ // NS


def _sc_aggr_body(h_hbm, src_hbm, dst_hbm, zero_hbm, out_hbm,
                  aggr, sidx, didx, rows0, rows1, sem0, sem1, semz):
    c = lax.axis_index("c")
    s = lax.axis_index("s")
    wid = c * NS + s
    row0 = pl.multiple_of(s * RPS, 8)
    zcp = pltpu.async_copy(zero_hbm.at[pl.ds(row0, RPS)],
                           aggr.at[pl.ds(row0, RPS)], semz)
    pltpu.sync_copy(src_hbm.at[wid, 0], sidx)
    pltpu.sync_copy(dst_hbm.at[wid, 0], didx)
    pltpu.sync_copy(h_hbm.at[sidx.at[0]], rows0)
    zcp.wait()
    plsc.subcore_barrier()

    for p in range(NP):

        @pl.loop(0, JP, step=2)
        def _(j):
            s0 = pltpu.async_copy(rows0, aggr.at[didx.at[j]], sem0,
                                  add=True)
            pltpu.sync_copy(h_hbm.at[sidx.at[j + 1]], rows1)
            s1 = pltpu.async_copy(rows1, aggr.at[didx.at[j + 1]], sem1,
                                  add=True)
            s0.wait()
            pltpu.sync_copy(h_hbm.at[sidx.at[j + 2]], rows0)
            s1.wait()

        if p + 1 < NP:
            pltpu.sync_copy(src_hbm.at[wid, p + 1], sidx)
            pltpu.sync_copy(dst_hbm.at[wid, p + 1], didx)

    plsc.subcore_barrier()
    pltpu.sync_copy(aggr.at[pl.ds(row0, RPS)],
                    out_hbm.at[c, pl.ds(row0, RPS)])


@jax.jit
def _sc_aggr(h, src_r, dst_r, zeros):
    mesh = plsc.VectorSubcoreMesh(core_axis_name="c", subcore_axis_name="s")
    return pl.kernel(
        _sc_aggr_body,
        out_type=jax.ShapeDtypeStruct((NC, NZ, D), jnp.float32),
        mesh=mesh,
        scratch_types=[
            pltpu.VMEM_SHARED((NZ, D), jnp.float32),
            pltpu.VMEM((JP + 1, EB), jnp.int32),
            pltpu.VMEM((JP + 1, EB), jnp.int32),
            pltpu.VMEM((EB, D), jnp.float32),
            pltpu.VMEM((EB, D), jnp.float32),
            pltpu.SemaphoreType.DMA,
            pltpu.SemaphoreType.DMA,
            pltpu.SemaphoreType.DMA,
        ],
    )(h, src_r, dst_r, zeros)


def _mlp_body(x_ref, a0_ref, a1_ref, w1_ref, b1_ref, w2_ref, b2_ref, o_ref):
    z = x_ref[...] + a0_ref[0] + a1_ref[0]
    z1 = jnp.maximum(
        jnp.dot(z, w1_ref[...], preferred_element_type=jnp.float32)
        + b1_ref[...], 0.0)
    o_ref[...] = (jnp.dot(z1, w2_ref[...], preferred_element_type=jnp.float32)
                  + b2_ref[...])


def _mlp_pool_body(x_ref, a0_ref, a1_ref, w1_ref, b1_ref, w2_ref, b2_ref,
                   batch_ref, o_ref):
    i = pl.program_id(0)
    z = x_ref[...] + a0_ref[0] + a1_ref[0]
    z1 = jnp.maximum(
        jnp.dot(z, w1_ref[...], preferred_element_type=jnp.float32)
        + b1_ref[...], 0.0)
    h3 = (jnp.dot(z1, w2_ref[...], preferred_element_type=jnp.float32)
          + b2_ref[...])
    ids = batch_ref[0, 0]
    mask = (jax.lax.broadcasted_iota(jnp.int32, (G, ids.shape[0]), 0)
            == ids[None, :]).astype(jnp.float32)
    pooled = jnp.dot(mask, h3, preferred_element_type=jnp.float32)

    @pl.when(i == 0)
    def _():
        o_ref[...] = jnp.zeros_like(o_ref)

    o_ref[...] += pooled


_RB = 2000
_NB = N // _RB

_TC_SPECS = [
    pl.BlockSpec((_RB, D), lambda i: (i, 0)),
    pl.BlockSpec((1, _RB, D), lambda i: (0, i, 0)),
    pl.BlockSpec((1, _RB, D), lambda i: (1, i, 0)),
    pl.BlockSpec((D, D), lambda i: (0, 0)),
    pl.BlockSpec((1, D), lambda i: (0, 0)),
    pl.BlockSpec((D, D), lambda i: (0, 0)),
    pl.BlockSpec((1, D), lambda i: (0, 0)),
]


@jax.jit
def _tc_mlp(h, parts, w1, b1, w2, b2):
    return pl.pallas_call(
        _mlp_body,
        grid=(_NB,),
        in_specs=_TC_SPECS,
        out_specs=pl.BlockSpec((_RB, D), lambda i: (i, 0)),
        out_shape=jax.ShapeDtypeStruct((N, D), jnp.float32),
    )(h, parts, parts, w1, b1.reshape(1, D), w2, b2.reshape(1, D))


@jax.jit
def _tc_mlp_pool(h, parts, w1, b1, w2, b2, batch_r):
    return pl.pallas_call(
        _mlp_pool_body,
        grid=(_NB,),
        in_specs=_TC_SPECS + [pl.BlockSpec((1, 1, _RB), lambda i: (i, 0, 0))],
        out_specs=pl.BlockSpec((G, D), lambda i: (0, 0)),
        out_shape=jax.ShapeDtypeStruct((G, D), jnp.float32),
    )(h, parts, parts, w1, b1.reshape(1, D), w2, b2.reshape(1, D), batch_r)


def _stage_idx(v, fills, tail):
    r = jnp.concatenate([v, fills]).reshape(NW, JW, EB)
    r = jnp.concatenate([r, tail.reshape(NW, 1, EB)], 1)
    passes = [r[:, p * JP:p * JP + JP + 1] for p in range(NP)]
    return jnp.stack(passes, axis=1)


def kernel(x, edge_index, batch, W1_0, b1_0, W2_0, b2_0, W1_1, b1_1, W2_1,
           b2_1, W1_2, b1_2, W2_2, b2_2):
    pad = jnp.arange(E_PAD - E, dtype=jnp.int32)
    tail = jnp.arange(NW * EB, dtype=jnp.int32)
    src_r = _stage_idx(edge_index[0], pad % N, tail % N)
    dst_r = _stage_idx(edge_index[1], N + pad % NTRASH, N + tail % NTRASH)
    zeros = jnp.zeros((NZ, D), jnp.float32)
    batch_r = batch.reshape(_NB, 1, _RB)

    params = [(W1_0, b1_0, W2_0, b2_0), (W1_1, b1_1, W2_1, b2_1),
              (W1_2, b1_2, W2_2, b2_2)]
    h = x
    out = None
    for l, (w1, b1, w2, b2) in enumerate(params):
        parts = _sc_aggr(h, src_r, dst_r, zeros)
        if l < 2:
            h = _tc_mlp(h, parts, w1, b1, w2, b2)
        else:
            out = _tc_mlp_pool(h, parts, w1, b1, w2, b2, batch_r)
    return out
